# baseline (speedup 1.0000x reference)
import functools

import jax
import jax.numpy as jnp
from jax import lax
from jax.experimental import pallas as pl

B = 1024
M = 100000
LOC = 128
SEN = 64
ACT = 8
HID = 256
G = 8

C = 16
BM = 2048
MPAD = 100352
NCH = MPAD // C
NBLK = MPAD // BM
CPB = BM // C
NEG = float("-inf")

_INTERPRET = False


def _pc(body, grid, in_specs, out_specs, out_shape):
    return pl.pallas_call(
        body, grid=grid, in_specs=in_specs, out_specs=out_specs,
        out_shape=out_shape, interpret=_INTERPRET)


def _k1_body(sens_ref, keys_ref, validf_ref, cm_ref, okeys_ref):
    i = pl.program_id(0)
    s = sens_ref[...]
    k = keys_ref[...]
    scores = lax.dot_general(s, k, (((1,), (1,)), ((), ())),
                             preferred_element_type=jnp.float32)
    masked = jnp.where(validf_ref[...] > 0.0, scores, NEG)
    acc = masked[:, 0:CPB]
    for t in range(1, C):
        acc = jnp.maximum(acc, masked[:, t * CPB:(t + 1) * CPB])
    cm_ref[...] = acc
    okeys_ref[...] = k

    @pl.when(i == 0)
    def _():
        okeys_ref[0:B, :] = s


def _run_k1(sensory, mem_keys, validf):
    return _pc(
        _k1_body, (NBLK,),
        [
            pl.BlockSpec((B, SEN), lambda i: (0, 0)),
            pl.BlockSpec((BM, SEN), lambda i: (i, 0)),
            pl.BlockSpec((1, BM), lambda i: (0, i)),
        ],
        [
            pl.BlockSpec((B, CPB), lambda i: (0, i)),
            pl.BlockSpec((BM, SEN), lambda i: (i, 0)),
        ],
        [
            jax.ShapeDtypeStruct((B, NCH), jnp.float32),
            jax.ShapeDtypeStruct((M, SEN), jnp.float32),
        ],
    )(sensory, mem_keys, validf)


_RB = 256


def _k2_body(cm_ref, ids_ref):
    x = cm_ref[...]
    iota = lax.broadcasted_iota(jnp.int32, x.shape, 1)
    cols = lax.broadcasted_iota(jnp.int32, (_RB, 128), 1)
    acc = jnp.zeros((_RB, 128), jnp.int32)
    for g in range(G):
        m = jnp.max(x, axis=1, keepdims=True)
        pos = jnp.min(jnp.where(x == m, iota, jnp.int32(2**30)),
                      axis=1, keepdims=True)
        acc = jnp.where(cols == g, pos, acc)
        x = jnp.where(iota == pos, NEG, x)
    ids_ref[...] = acc


def _run_k2(cm):
    return _pc(
        _k2_body, (B // _RB,),
        [pl.BlockSpec((_RB, NCH), lambda i: (i, 0))],
        pl.BlockSpec((_RB, 128), lambda i: (i, 0)),
        jax.ShapeDtypeStruct((B, 128), jnp.int32),
    )(cm)


_RB4 = 128


def _k4_body(ck_ref, cv_ref, gid_ref, sens_ref, sel_ref):
    ck = ck_ref[...].reshape(_RB4, G * C, SEN)
    s = sens_ref[...][:, None, :]
    cs = jnp.sum(ck * s, axis=-1)
    gids = gid_ref[...]
    ok = (cv_ref[...] > 0.0) & (gids < M)
    cs = jnp.where(ok, cs, NEG)
    cols = lax.broadcasted_iota(jnp.int32, (_RB4, 128), 1)
    acc = jnp.zeros((_RB4, 128), jnp.int32)
    fcnt = jnp.zeros((_RB4, 1), jnp.int32)
    for g in range(G):
        m = jnp.max(cs, axis=1, keepdims=True)
        gid = jnp.min(jnp.where(cs == m, gids, jnp.int32(2**30)),
                      axis=1, keepdims=True)
        acc = jnp.where(cols == g, gid, acc)
        fcnt = fcnt + (m > NEG).astype(jnp.int32)
        cs = jnp.where(gids == gid, NEG, cs)
    acc = jnp.where(cols == G, fcnt, acc)
    sel_ref[...] = acc


def _run_k4(candkeys2d, candvalid, gids, sensory):
    return _pc(
        _k4_body, (B // _RB4,),
        [
            pl.BlockSpec((_RB4 * G * C, SEN), lambda i: (i, 0)),
            pl.BlockSpec((_RB4, G * C), lambda i: (i, 0)),
            pl.BlockSpec((_RB4, G * C), lambda i: (i, 0)),
            pl.BlockSpec((_RB4, SEN), lambda i: (i, 0)),
        ],
        pl.BlockSpec((_RB4, 128), lambda i: (i, 0)),
        jax.ShapeDtypeStruct((B, 128), jnp.int32),
    )(candkeys2d, candvalid, gids, sensory)


def _k6_body(loc_ref, act_ref, hdg_ref, w1a_ref, w1b_ref, b1_ref, w2_ref,
             b2_ref, v1a_ref, v1b_ref, v2_ref, v2b_ref, vs_ref, n1_ref,
             n2_ref, gss_ref, sel_ref, nloc_ref, obj_ref, lp_ref):
    f32 = jnp.float32

    def mm(a, b):
        return lax.dot_general(a, b, (((1,), (0,)), ((), ())),
                               preferred_element_type=f32)

    w1a, w1b, b1 = w1a_ref[...], w1b_ref[...], b1_ref[...]
    w2, b2 = w2_ref[...], b2_ref[...]
    v1a, v1b = v1a_ref[...], v1b_ref[...]
    v2, v2b = v2_ref[...], v2b_ref[...]
    vsc = vs_ref[...]

    def localizer(loc, act):
        h = jnp.maximum(mm(loc, w1a) + mm(act, w1b) + b1, 0.0)
        mean = mm(h, w2) + b2
        g = jnp.maximum(mm(mean, v1a) + v1b, 0.0)
        sd = jnp.exp(mm(g, v2) + v2b)
        sd = jnp.exp(vsc) * sd + 1e-6
        return mean, sd

    loc, act, hdg = loc_ref[...], act_ref[...], hdg_ref[...]
    n1, n2 = n1_ref[...], n2_ref[...]

    nl_mean, nl_sd = localizer(loc, act)
    nloc = nl_mean + n1 * nl_sd
    nloc_ref[...] = nloc
    nl_q = jnp.sum(((nloc - nl_mean) / nl_sd) ** 2, axis=-1, keepdims=True)
    nl_lp = -0.5 * jnp.mean(nl_q) - jnp.mean(
        jnp.sum(jnp.log(nl_sd), axis=-1, keepdims=True))

    exp_mean, exp_sd = localizer(nloc, hdg)

    found = sel_ref[...][:, G:G + 1]
    ignore = found == 0

    gss = gss_ref[...]
    esph = exp_mean / (jnp.sqrt(jnp.sum(exp_mean * exp_mean, axis=-1,
                                        keepdims=True)) + 1e-8)
    gnrm = jnp.sqrt(jnp.sum(gss * gss, axis=-1, keepdims=True))
    gsph = gss / (gnrm + 1e-8)
    matches = jnp.sum(gsph * esph[:, None, :], axis=-1)
    slot = lax.broadcasted_iota(jnp.int32, (B, G), 1)
    invalid = slot >= found
    matches = jnp.where(invalid, NEG, matches)
    mmax = jnp.max(matches, axis=-1, keepdims=True)
    unnorm = jnp.exp(matches - mmax)
    w = unnorm / jnp.sum(unnorm, axis=-1, keepdims=True)
    integrated = jnp.sum(w[:, :, None] * gss, axis=1)

    dz = -0.5 * ((integrated - exp_mean) / exp_sd)
    too_far = jnp.sqrt(jnp.sum(dz * dz, axis=-1, keepdims=True)) > 2.0
    expected = exp_mean + n2 * exp_sd
    cond = too_far | ignore | jnp.isnan(integrated)
    obj = jnp.where(cond, expected, integrated)
    obj_ref[...] = obj

    ob_q = jnp.sum(((obj - exp_mean) / exp_sd) ** 2, axis=-1, keepdims=True)
    obj_lp = -0.5 * jnp.mean(ob_q) - jnp.mean(
        jnp.sum(jnp.log(exp_sd), axis=-1, keepdims=True))
    lp_ref[...] = jnp.full((1, 128), nl_lp + obj_lp, f32)


def _run_k6(last_location, action, heading, W1, b1, W2, b2, V1, v1, V2, v2,
            vscale, noise1, noise2, guesses, sel):
    full = lambda s: pl.BlockSpec(s, lambda: tuple(0 for _ in s))
    args = (last_location, action, heading, W1[:LOC], W1[LOC:],
            b1.reshape(1, HID), W2, b2.reshape(1, LOC), V1[:LOC],
            v1.reshape(1, HID), V2, v2.reshape(1, LOC),
            vscale.reshape(1, LOC), noise1, noise2, guesses, sel)
    return _pc(
        _k6_body, (),
        [full(a.shape) for a in args],
        [full((B, LOC)), full((B, LOC)), full((1, 128))],
        [
            jax.ShapeDtypeStruct((B, LOC), jnp.float32),
            jax.ShapeDtypeStruct((B, LOC), jnp.float32),
            jax.ShapeDtypeStruct((1, 128), jnp.float32),
        ],
    )(*args)


_VB = 4000


def _k7_body(mv_ref, obj_ref, out_ref):
    i = pl.program_id(0)
    out_ref[...] = mv_ref[...]

    @pl.when(i == 0)
    def _():
        out_ref[0:B, :] = obj_ref[...]


def _run_k7(mem_values, obj):
    return _pc(
        _k7_body, (M // _VB,),
        [
            pl.BlockSpec((_VB, LOC), lambda i: (i, 0)),
            pl.BlockSpec((B, LOC), lambda i: (0, 0)),
        ],
        pl.BlockSpec((_VB, LOC), lambda i: (i, 0)),
        jax.ShapeDtypeStruct((M, LOC), jnp.float32),
    )(mem_values, obj)


def kernel(sensory, last_location, action, heading, W1, b1, W2, b2, V1, v1,
           V2, v2, vscale, mem_keys, mem_values, mem_valid):
    validf = jnp.pad(mem_valid.astype(jnp.float32), (0, MPAD - M))

    cm, new_mem_keys = _run_k1(sensory, mem_keys, validf.reshape(1, MPAD))
    ids = _run_k2(cm)

    chunk_ids = ids[:, :G]
    blk, j = chunk_ids // CPB, chunk_ids % CPB
    gids = ((blk * BM + j)[:, :, None] +
            CPB * jnp.arange(C, dtype=jnp.int32)).reshape(B, G * C)
    gidc = jnp.minimum(gids, M - 1)
    candkeys2d = jnp.take(mem_keys, gidc.reshape(-1), axis=0)
    candvalid = jnp.take(validf, gidc.reshape(-1)).reshape(B, G * C)

    sel = _run_k4(candkeys2d, candvalid, gids, sensory)

    fidx = jnp.minimum(sel[:, :G], M - 1)
    guesses = jnp.take(mem_values, fidx.reshape(-1),
                       axis=0).reshape(B, G, LOC)

    noise1 = jax.random.normal(jax.random.fold_in(jax.random.key(42), 1),
                               (B, LOC), jnp.float32)
    noise2 = jax.random.normal(jax.random.fold_in(jax.random.key(42), 2),
                               (B, LOC), jnp.float32)

    next_location, obj, lp = _run_k6(
        last_location, action, heading, W1, b1, W2, b2, V1, v1, V2, v2,
        vscale, noise1, noise2, guesses, sel)

    new_mem_values = _run_k7(mem_values, obj)
    return (next_location, lp[0, 0], new_mem_keys, new_mem_values)


# speedup vs baseline: 1.0278x; 1.0278x over previous
import functools

import jax
import jax.numpy as jnp
from jax import lax
from jax.experimental import pallas as pl
from jax.experimental.pallas import tpu as pltpu
from jax.experimental.pallas import tpu_sc as plsc

B = 1024
M = 100000
LOC = 128
SEN = 64
ACT = 8
HID = 256
G = 8

C = 16
BM = 2048
MPAD = 100352
NCH = MPAD // C
NBLK = MPAD // BM
CPB = BM // C
NEG = float("-inf")

_INTERPRET = False


def _pc(body, grid, in_specs, out_specs, out_shape):
    return pl.pallas_call(
        body, grid=grid, in_specs=in_specs, out_specs=out_specs,
        out_shape=out_shape, interpret=_INTERPRET)


def _k1_body(sens_ref, keys_ref, validf_ref, cm_ref, okeys_ref):
    i = pl.program_id(0)
    s = sens_ref[...]
    k = keys_ref[...]
    scores = lax.dot_general(s, k, (((1,), (1,)), ((), ())),
                             preferred_element_type=jnp.float32)
    masked = jnp.where(validf_ref[...] > 0.0, scores, NEG)
    acc = masked[:, 0:CPB]
    for t in range(1, C):
        acc = jnp.maximum(acc, masked[:, t * CPB:(t + 1) * CPB])
    cm_ref[...] = acc
    okeys_ref[...] = k

    @pl.when(i == 0)
    def _():
        okeys_ref[0:B, :] = s


def _run_k1(sensory, mem_keys, validf):
    return _pc(
        _k1_body, (NBLK,),
        [
            pl.BlockSpec((B, SEN), lambda i: (0, 0)),
            pl.BlockSpec((BM, SEN), lambda i: (i, 0)),
            pl.BlockSpec((1, BM), lambda i: (0, i)),
        ],
        [
            pl.BlockSpec((B, CPB), lambda i: (0, i)),
            pl.BlockSpec((BM, SEN), lambda i: (i, 0)),
        ],
        [
            jax.ShapeDtypeStruct((B, NCH), jnp.float32),
            jax.ShapeDtypeStruct((M, SEN), jnp.float32),
        ],
    )(sensory, mem_keys, validf)


_RB = 256


def _k2_body(cm_ref, ids_ref):
    x = cm_ref[...]
    iota = lax.broadcasted_iota(jnp.int32, x.shape, 1)
    cols = lax.broadcasted_iota(jnp.int32, (_RB, 128), 1)
    acc = jnp.zeros((_RB, 128), jnp.int32)
    for g in range(G):
        m = jnp.max(x, axis=1, keepdims=True)
        hit = x == m
        pos = jnp.min(jnp.where(hit, iota, jnp.int32(2**30)),
                      axis=1, keepdims=True)
        acc = jnp.where(cols == g, pos, acc)
        x = jnp.where(hit, NEG, x)
    ids_ref[...] = acc


def _run_k2(cm):
    return _pc(
        _k2_body, (B // _RB,),
        [pl.BlockSpec((_RB, NCH), lambda i: (i, 0))],
        pl.BlockSpec((_RB, 128), lambda i: (i, 0)),
        jax.ShapeDtypeStruct((B, 128), jnp.int32),
    )(cm)


_RB4 = 128


def _k4_body(ck_ref, cv_ref, gid_ref, sens_ref, sel_ref):
    ck = ck_ref[...].reshape(_RB4, G * C, SEN)
    s = sens_ref[...][:, None, :]
    cs = jnp.sum(ck * s, axis=-1)
    gids = gid_ref[...]
    ok = (cv_ref[...] > 0.0) & (gids < M)
    cs = jnp.where(ok, cs, NEG)
    cols = lax.broadcasted_iota(jnp.int32, (_RB4, 128), 1)
    acc = jnp.zeros((_RB4, 128), jnp.int32)
    fcnt = jnp.zeros((_RB4, 1), jnp.int32)
    for g in range(G):
        m = jnp.max(cs, axis=1, keepdims=True)
        hit = cs == m
        gid = jnp.min(jnp.where(hit, gids, jnp.int32(2**30)),
                      axis=1, keepdims=True)
        acc = jnp.where(cols == g, gid, acc)
        fcnt = fcnt + (m > NEG).astype(jnp.int32)
        cs = jnp.where(hit, NEG, cs)
    acc = jnp.where(cols == G, fcnt, acc)
    sel_ref[...] = acc


def _run_k4(candkeys2d, candvalid, gids, sensory):
    return _pc(
        _k4_body, (B // _RB4,),
        [
            pl.BlockSpec((_RB4 * G * C, SEN), lambda i: (i, 0)),
            pl.BlockSpec((_RB4, G * C), lambda i: (i, 0)),
            pl.BlockSpec((_RB4, G * C), lambda i: (i, 0)),
            pl.BlockSpec((_RB4, SEN), lambda i: (i, 0)),
        ],
        pl.BlockSpec((_RB4, 128), lambda i: (i, 0)),
        jax.ShapeDtypeStruct((B, 128), jnp.int32),
    )(candkeys2d, candvalid, gids, sensory)


def _k6_body(loc_ref, act_ref, hdg_ref, w1a_ref, w1b_ref, b1_ref, w2_ref,
             b2_ref, v1a_ref, v1b_ref, v2_ref, v2b_ref, vs_ref, n1_ref,
             n2_ref, gss_ref, sel_ref, nloc_ref, obj_ref, lp_ref):
    f32 = jnp.float32

    def mm(a, b):
        return lax.dot_general(a, b, (((1,), (0,)), ((), ())),
                               preferred_element_type=f32)

    w1a, w1b, b1 = w1a_ref[...], w1b_ref[...], b1_ref[...]
    w2, b2 = w2_ref[...], b2_ref[...]
    v1a, v1b = v1a_ref[...], v1b_ref[...]
    v2, v2b = v2_ref[...], v2b_ref[...]
    vsc = vs_ref[...]

    def localizer(loc, act):
        h = jnp.maximum(mm(loc, w1a) + mm(act, w1b) + b1, 0.0)
        mean = mm(h, w2) + b2
        g = jnp.maximum(mm(mean, v1a) + v1b, 0.0)
        sd = jnp.exp(mm(g, v2) + v2b)
        sd = jnp.exp(vsc) * sd + 1e-6
        return mean, sd

    loc, act, hdg = loc_ref[...], act_ref[...], hdg_ref[...]
    n1, n2 = n1_ref[...], n2_ref[...]

    nl_mean, nl_sd = localizer(loc, act)
    nloc = nl_mean + n1 * nl_sd
    nloc_ref[...] = nloc
    nl_q = jnp.sum(((nloc - nl_mean) / nl_sd) ** 2, axis=-1, keepdims=True)
    nl_lp = -0.5 * jnp.mean(nl_q) - jnp.mean(
        jnp.sum(jnp.log(nl_sd), axis=-1, keepdims=True))

    exp_mean, exp_sd = localizer(nloc, hdg)

    found = sel_ref[...][:, G:G + 1]
    ignore = found == 0

    gss = gss_ref[...]
    esph = exp_mean / (jnp.sqrt(jnp.sum(exp_mean * exp_mean, axis=-1,
                                        keepdims=True)) + 1e-8)
    gnrm = jnp.sqrt(jnp.sum(gss * gss, axis=-1, keepdims=True))
    gsph = gss / (gnrm + 1e-8)
    matches = jnp.sum(gsph * esph[:, None, :], axis=-1)
    slot = lax.broadcasted_iota(jnp.int32, (B, G), 1)
    invalid = slot >= found
    matches = jnp.where(invalid, NEG, matches)
    mmax = jnp.max(matches, axis=-1, keepdims=True)
    unnorm = jnp.exp(matches - mmax)
    w = unnorm / jnp.sum(unnorm, axis=-1, keepdims=True)
    integrated = jnp.sum(w[:, :, None] * gss, axis=1)

    dz = -0.5 * ((integrated - exp_mean) / exp_sd)
    too_far = jnp.sqrt(jnp.sum(dz * dz, axis=-1, keepdims=True)) > 2.0
    expected = exp_mean + n2 * exp_sd
    cond = too_far | ignore | jnp.isnan(integrated)
    obj = jnp.where(cond, expected, integrated)
    obj_ref[...] = obj

    ob_q = jnp.sum(((obj - exp_mean) / exp_sd) ** 2, axis=-1, keepdims=True)
    obj_lp = -0.5 * jnp.mean(ob_q) - jnp.mean(
        jnp.sum(jnp.log(exp_sd), axis=-1, keepdims=True))
    lp_ref[...] = jnp.full((1, 128), nl_lp + obj_lp, f32)


def _run_k6(last_location, action, heading, W1, b1, W2, b2, V1, v1, V2, v2,
            vscale, noise1, noise2, guesses, sel):
    full = lambda s: pl.BlockSpec(s, lambda: tuple(0 for _ in s))
    args = (last_location, action, heading, W1[:LOC], W1[LOC:],
            b1.reshape(1, HID), W2, b2.reshape(1, LOC), V1[:LOC],
            v1.reshape(1, HID), V2, v2.reshape(1, LOC),
            vscale.reshape(1, LOC), noise1, noise2, guesses, sel)
    return _pc(
        _k6_body, (),
        [full(a.shape) for a in args],
        [full((B, LOC)), full((B, LOC)), full((1, 128))],
        [
            jax.ShapeDtypeStruct((B, LOC), jnp.float32),
            jax.ShapeDtypeStruct((B, LOC), jnp.float32),
            jax.ShapeDtypeStruct((1, 128), jnp.float32),
        ],
    )(*args)


_VB = 4000


def _k7_body(mv_ref, obj_ref, out_ref):
    i = pl.program_id(0)
    out_ref[...] = mv_ref[...]

    @pl.when(i == 0)
    def _():
        out_ref[0:B, :] = obj_ref[...]


def _run_k7(mem_values, obj):
    return _pc(
        _k7_body, (M // _VB,),
        [
            pl.BlockSpec((_VB, LOC), lambda i: (i, 0)),
            pl.BlockSpec((B, LOC), lambda i: (0, 0)),
        ],
        pl.BlockSpec((_VB, LOC), lambda i: (i, 0)),
        jax.ShapeDtypeStruct((M, LOC), jnp.float32),
    )(mem_values, obj)


_NW = 32


def _sc_gather(table, idx, ncols, chunk):
    nidx = idx.shape[0]
    per_w = nidx // _NW
    mesh = plsc.VectorSubcoreMesh(core_axis_name="c", subcore_axis_name="s")

    @functools.partial(
        pl.kernel, mesh=mesh,
        out_type=jax.ShapeDtypeStruct((nidx, ncols), jnp.float32),
        scratch_types=[
            pltpu.VMEM((chunk,), jnp.int32),
            pltpu.VMEM((chunk, ncols), jnp.float32),
            pltpu.SemaphoreType.DMA,
        ],
    )
    def gk(table_hbm, idx_hbm, out_hbm, idx_v, rows_v, sem):
        wid = lax.axis_index("s") * 2 + lax.axis_index("c")
        base = wid * per_w
        for b in range(per_w // chunk):
            off = base + b * chunk
            pltpu.sync_copy(idx_hbm.at[pl.ds(off, chunk)], idx_v)
            pltpu.async_copy(table_hbm.at[idx_v], rows_v, sem).wait()
            pltpu.sync_copy(rows_v, out_hbm.at[pl.ds(off, chunk)])

    return gk(table, idx)


def kernel(sensory, last_location, action, heading, W1, b1, W2, b2, V1, v1,
           V2, v2, vscale, mem_keys, mem_values, mem_valid):
    validf = jnp.pad(mem_valid.astype(jnp.float32), (0, MPAD - M))

    cm, new_mem_keys = _run_k1(sensory, mem_keys, validf.reshape(1, MPAD))
    ids = _run_k2(cm)

    chunk_ids = ids[:, :G]
    blk, j = chunk_ids // CPB, chunk_ids % CPB
    gids = ((blk * BM + j)[:, :, None] +
            CPB * jnp.arange(C, dtype=jnp.int32)).reshape(B, G * C)
    gidc = jnp.minimum(gids, M - 1)
    candkeys2d = jnp.take(mem_keys, gidc.reshape(-1), axis=0)
    candvalid = jnp.take(validf, gidc.reshape(-1)).reshape(B, G * C)

    sel = _run_k4(candkeys2d, candvalid, gids, sensory)

    fidx = jnp.minimum(sel[:, :G], M - 1)
    guesses = _sc_gather(mem_values, fidx.reshape(-1),
                         LOC, B * G // _NW).reshape(B, G, LOC)

    noise1 = jax.random.normal(jax.random.fold_in(jax.random.key(42), 1),
                               (B, LOC), jnp.float32)
    noise2 = jax.random.normal(jax.random.fold_in(jax.random.key(42), 2),
                               (B, LOC), jnp.float32)

    next_location, obj, lp = _run_k6(
        last_location, action, heading, W1, b1, W2, b2, V1, v1, V2, v2,
        vscale, noise1, noise2, guesses, sel)

    new_mem_values = _run_k7(mem_values, obj)
    return (next_location, lp[0, 0], new_mem_keys, new_mem_values)


# speedup vs baseline: 1.1570x; 1.1257x over previous
import functools

import jax
import jax.numpy as jnp
from jax import lax
from jax.experimental import pallas as pl
from jax.experimental.pallas import tpu as pltpu
from jax.experimental.pallas import tpu_sc as plsc

B = 1024
M = 100000
LOC = 128
SEN = 64
ACT = 8
HID = 256
G = 8

C = 16
BM = 2048
MPAD = 100352
NCH = MPAD // C
NBLK = MPAD // BM
CPB = BM // C
NEG = float("-inf")

_INTERPRET = False


def _pc(body, grid, in_specs, out_specs, out_shape):
    return pl.pallas_call(
        body, grid=grid, in_specs=in_specs, out_specs=out_specs,
        out_shape=out_shape, interpret=_INTERPRET)


def _k1_body(sens_ref, keys_ref, validf_ref, cm_ref, okeys_ref, keys2_ref):
    i = pl.program_id(0)
    s = sens_ref[...]
    k = keys_ref[...]
    scores = lax.dot_general(s, k, (((1,), (1,)), ((), ())),
                             preferred_element_type=jnp.float32)
    v = validf_ref[...]
    masked = jnp.where(v > 0.0, scores, NEG)
    acc = masked[:, 0:CPB]
    for t in range(1, C):
        acc = jnp.maximum(acc, masked[:, t * CPB:(t + 1) * CPB])
    cm_ref[...] = acc
    okeys_ref[...] = k

    bias_col = jnp.transpose(jnp.where(v > 0.0, 0.0, -1e30))
    keys2_ref[...] = jnp.concatenate(
        [k, bias_col, jnp.zeros((BM, LOC - SEN - 1), jnp.float32)], axis=1)

    @pl.when(i == 0)
    def _():
        okeys_ref[0:B, :] = s


def _run_k1(sensory, mem_keys, validf):
    return _pc(
        _k1_body, (NBLK,),
        [
            pl.BlockSpec((B, SEN), lambda i: (0, 0)),
            pl.BlockSpec((BM, SEN), lambda i: (i, 0)),
            pl.BlockSpec((1, BM), lambda i: (0, i)),
        ],
        [
            pl.BlockSpec((B, CPB), lambda i: (0, i)),
            pl.BlockSpec((BM, SEN), lambda i: (i, 0)),
            pl.BlockSpec((BM, LOC), lambda i: (i, 0)),
        ],
        [
            jax.ShapeDtypeStruct((B, NCH), jnp.float32),
            jax.ShapeDtypeStruct((M, SEN), jnp.float32),
            jax.ShapeDtypeStruct((M, LOC), jnp.float32),
        ],
    )(sensory, mem_keys, validf)


_RB = 256


def _k2_body(cm_ref, ids_ref):
    x = cm_ref[...]
    iota = lax.broadcasted_iota(jnp.int32, x.shape, 1)
    cols = lax.broadcasted_iota(jnp.int32, (_RB, 128), 1)
    acc = jnp.zeros((_RB, 128), jnp.int32)
    for g in range(G):
        m = jnp.max(x, axis=1, keepdims=True)
        hit = x == m
        pos = jnp.min(jnp.where(hit, iota, jnp.int32(2**30)),
                      axis=1, keepdims=True)
        acc = jnp.where(cols == g, pos, acc)
        x = jnp.where(hit, NEG, x)
    ids_ref[...] = acc


def _run_k2(cm):
    return _pc(
        _k2_body, (B // _RB,),
        [pl.BlockSpec((_RB, NCH), lambda i: (i, 0))],
        pl.BlockSpec((_RB, 128), lambda i: (i, 0)),
        jax.ShapeDtypeStruct((B, 128), jnp.int32),
    )(cm)


_RB4 = 128


def _k4_body(ck_ref, gid_ref, sens_ref, sel_ref):
    ck = ck_ref[...].reshape(_RB4, G * C, LOC)
    s = sens_ref[...]
    s128 = jnp.concatenate(
        [s, jnp.ones((_RB4, 1), jnp.float32),
         jnp.zeros((_RB4, LOC - SEN - 1), jnp.float32)], axis=1)
    cs = jnp.sum(ck * s128[:, None, :], axis=-1)
    gids = gid_ref[...]
    cs = jnp.where(gids < M, cs, NEG)
    cols = lax.broadcasted_iota(jnp.int32, (_RB4, 128), 1)
    acc = jnp.zeros((_RB4, 128), jnp.int32)
    fcnt = jnp.zeros((_RB4, 1), jnp.int32)
    for g in range(G):
        m = jnp.max(cs, axis=1, keepdims=True)
        hit = cs == m
        gid = jnp.min(jnp.where(hit, gids, jnp.int32(2**30)),
                      axis=1, keepdims=True)
        acc = jnp.where(cols == g, gid, acc)
        fcnt = fcnt + (m > -1e29).astype(jnp.int32)
        cs = jnp.where(hit, NEG, cs)
    acc = jnp.where(cols == G, fcnt, acc)
    sel_ref[...] = acc


def _run_k4(candkeys2d, gids, sensory):
    return _pc(
        _k4_body, (B // _RB4,),
        [
            pl.BlockSpec((_RB4 * G * C, LOC), lambda i: (i, 0)),
            pl.BlockSpec((_RB4, G * C), lambda i: (i, 0)),
            pl.BlockSpec((_RB4, SEN), lambda i: (i, 0)),
        ],
        pl.BlockSpec((_RB4, 128), lambda i: (i, 0)),
        jax.ShapeDtypeStruct((B, 128), jnp.int32),
    )(candkeys2d, gids, sensory)


def _k6_body(loc_ref, act_ref, hdg_ref, w1a_ref, w1b_ref, b1_ref, w2_ref,
             b2_ref, v1a_ref, v1b_ref, v2_ref, v2b_ref, vs_ref, n1_ref,
             n2_ref, gss_ref, sel_ref, nloc_ref, obj_ref, lp_ref):
    f32 = jnp.float32

    def mm(a, b):
        return lax.dot_general(a, b, (((1,), (0,)), ((), ())),
                               preferred_element_type=f32)

    w1a, w1b, b1 = w1a_ref[...], w1b_ref[...], b1_ref[...]
    w2, b2 = w2_ref[...], b2_ref[...]
    v1a, v1b = v1a_ref[...], v1b_ref[...]
    v2, v2b = v2_ref[...], v2b_ref[...]
    vsc = vs_ref[...]

    def localizer(loc, act):
        h = jnp.maximum(mm(loc, w1a) + mm(act, w1b) + b1, 0.0)
        mean = mm(h, w2) + b2
        g = jnp.maximum(mm(mean, v1a) + v1b, 0.0)
        sd = jnp.exp(mm(g, v2) + v2b)
        sd = jnp.exp(vsc) * sd + 1e-6
        return mean, sd

    loc, act, hdg = loc_ref[...], act_ref[...], hdg_ref[...]
    n1, n2 = n1_ref[...], n2_ref[...]

    nl_mean, nl_sd = localizer(loc, act)
    nloc = nl_mean + n1 * nl_sd
    nloc_ref[...] = nloc
    nl_q = jnp.sum(((nloc - nl_mean) / nl_sd) ** 2, axis=-1, keepdims=True)
    nl_lp = -0.5 * jnp.mean(nl_q) - jnp.mean(
        jnp.sum(jnp.log(nl_sd), axis=-1, keepdims=True))

    exp_mean, exp_sd = localizer(nloc, hdg)

    found = sel_ref[...][:, G:G + 1]
    ignore = found == 0

    gss = gss_ref[...]
    esph = exp_mean / (jnp.sqrt(jnp.sum(exp_mean * exp_mean, axis=-1,
                                        keepdims=True)) + 1e-8)
    gnrm = jnp.sqrt(jnp.sum(gss * gss, axis=-1, keepdims=True))
    gsph = gss / (gnrm + 1e-8)
    matches = jnp.sum(gsph * esph[:, None, :], axis=-1)
    slot = lax.broadcasted_iota(jnp.int32, (B, G), 1)
    invalid = slot >= found
    matches = jnp.where(invalid, NEG, matches)
    mmax = jnp.max(matches, axis=-1, keepdims=True)
    unnorm = jnp.exp(matches - mmax)
    w = unnorm / jnp.sum(unnorm, axis=-1, keepdims=True)
    integrated = jnp.sum(w[:, :, None] * gss, axis=1)

    dz = -0.5 * ((integrated - exp_mean) / exp_sd)
    too_far = jnp.sqrt(jnp.sum(dz * dz, axis=-1, keepdims=True)) > 2.0
    expected = exp_mean + n2 * exp_sd
    cond = too_far | ignore | jnp.isnan(integrated)
    obj = jnp.where(cond, expected, integrated)
    obj_ref[...] = obj

    ob_q = jnp.sum(((obj - exp_mean) / exp_sd) ** 2, axis=-1, keepdims=True)
    obj_lp = -0.5 * jnp.mean(ob_q) - jnp.mean(
        jnp.sum(jnp.log(exp_sd), axis=-1, keepdims=True))
    lp_ref[...] = jnp.full((1, 128), nl_lp + obj_lp, f32)


def _run_k6(last_location, action, heading, W1, b1, W2, b2, V1, v1, V2, v2,
            vscale, noise1, noise2, guesses, sel):
    full = lambda s: pl.BlockSpec(s, lambda: tuple(0 for _ in s))
    args = (last_location, action, heading, W1[:LOC], W1[LOC:],
            b1.reshape(1, HID), W2, b2.reshape(1, LOC), V1[:LOC],
            v1.reshape(1, HID), V2, v2.reshape(1, LOC),
            vscale.reshape(1, LOC), noise1, noise2, guesses, sel)
    return _pc(
        _k6_body, (),
        [full(a.shape) for a in args],
        [full((B, LOC)), full((B, LOC)), full((1, 128))],
        [
            jax.ShapeDtypeStruct((B, LOC), jnp.float32),
            jax.ShapeDtypeStruct((B, LOC), jnp.float32),
            jax.ShapeDtypeStruct((1, 128), jnp.float32),
        ],
    )(*args)


_VB = 4000


def _k7_body(mv_ref, obj_ref, out_ref):
    i = pl.program_id(0)
    out_ref[...] = mv_ref[...]

    @pl.when(i == 0)
    def _():
        out_ref[0:B, :] = obj_ref[...]


def _run_k7(mem_values, obj):
    return _pc(
        _k7_body, (M // _VB,),
        [
            pl.BlockSpec((_VB, LOC), lambda i: (i, 0)),
            pl.BlockSpec((B, LOC), lambda i: (0, 0)),
        ],
        pl.BlockSpec((_VB, LOC), lambda i: (i, 0)),
        jax.ShapeDtypeStruct((M, LOC), jnp.float32),
    )(mem_values, obj)


_NW = 32


def _sc_gather(table, idx, ncols, chunk):
    nidx = idx.shape[0]
    per_w = nidx // _NW
    mesh = plsc.VectorSubcoreMesh(core_axis_name="c", subcore_axis_name="s")

    @functools.partial(
        pl.kernel, mesh=mesh,
        out_type=jax.ShapeDtypeStruct((nidx, ncols), jnp.float32),
        scratch_types=[
            pltpu.VMEM((chunk,), jnp.int32),
            pltpu.VMEM((chunk, ncols), jnp.float32),
            pltpu.SemaphoreType.DMA,
        ],
    )
    def gk(table_hbm, idx_hbm, out_hbm, idx_v, rows_v, sem):
        wid = lax.axis_index("s") * 2 + lax.axis_index("c")
        base = wid * per_w
        for b in range(per_w // chunk):
            off = base + b * chunk
            pltpu.sync_copy(idx_hbm.at[pl.ds(off, chunk)], idx_v)
            pltpu.async_copy(table_hbm.at[idx_v], rows_v, sem).wait()
            pltpu.sync_copy(rows_v, out_hbm.at[pl.ds(off, chunk)])

    return gk(table, idx)


def kernel(sensory, last_location, action, heading, W1, b1, W2, b2, V1, v1,
           V2, v2, vscale, mem_keys, mem_values, mem_valid):
    validf = jnp.pad(mem_valid.astype(jnp.float32), (0, MPAD - M))

    cm, new_mem_keys, keys2 = _run_k1(sensory, mem_keys,
                                      validf.reshape(1, MPAD))
    ids = _run_k2(cm)

    chunk_ids = ids[:, :G]
    blk, j = chunk_ids // CPB, chunk_ids % CPB
    gids = ((blk * BM + j)[:, :, None] +
            CPB * jnp.arange(C, dtype=jnp.int32)).reshape(B, G * C)
    gidc = jnp.minimum(gids, M - 1)
    candkeys2d = _sc_gather(keys2, gidc.reshape(-1), LOC, 512)

    sel = _run_k4(candkeys2d, gids, sensory)

    fidx = jnp.minimum(sel[:, :G], M - 1)
    guesses = _sc_gather(mem_values, fidx.reshape(-1),
                         LOC, B * G // _NW).reshape(B, G, LOC)

    noise1 = jax.random.normal(jax.random.fold_in(jax.random.key(42), 1),
                               (B, LOC), jnp.float32)
    noise2 = jax.random.normal(jax.random.fold_in(jax.random.key(42), 2),
                               (B, LOC), jnp.float32)

    next_location, obj, lp = _run_k6(
        last_location, action, heading, W1, b1, W2, b2, V1, v1, V2, v2,
        vscale, noise1, noise2, guesses, sel)

    new_mem_values = _run_k7(mem_values, obj)
    return (next_location, lp[0, 0], new_mem_keys, new_mem_values)
